# Initial kernel scaffold; baseline (speedup 1.0000x reference)
import jax
import jax.numpy as jnp
from jax.experimental import pallas as pl


def kernel(inputs, targets, cams, proxy, pids, cids, D_cam):
    raise NotImplementedError("write your pallas kernel here")


# speedup vs baseline: 2.5715x; 2.5715x over previous
import functools

import jax
import jax.numpy as jnp
import numpy as np
from jax.experimental import pallas as pl

_F = 256
_N = 8192
_NH = 50
_TEMP = 0.07
_LAM_DIS = 0.05
_LAM_INS = 0.05
_GAMMA = 0.9
_NK = 5
_MC = 8
_B = 64


def _acm_main_kernel(x_ref, t_ref, c_ref, proxy_ref, p_ref, q_ref,
                     loss_ref, gram_ref, rowmeans_ref):
    x = x_ref[...]
    nrm = jnp.sqrt(jnp.sum(x * x, axis=1, keepdims=True))
    xn = x / jnp.maximum(nrm, 1e-12)
    proxy = proxy_ref[...]
    ip = jax.lax.dot_general(xn, proxy, (((1,), (1,)), ((), ())),
                             preferred_element_type=jnp.float32)
    gram = jax.lax.dot_general(xn, xn, (((1,), (1,)), ((), ())),
                               preferred_element_type=jnp.float32)
    gram_ref[...] = gram

    t = t_ref[...]
    c = c_ref[...]
    p = p_ref[...]
    q = q_ref[...]

    sims = ip * (1.0 / _TEMP)
    pos = (t == p) & (c != q)
    neg = t != p
    npos = jnp.sum(pos.astype(jnp.float32), axis=1, keepdims=True)
    masked = sims + jnp.where(neg, 0.0, -9999999.0)

    iota = jax.lax.broadcasted_iota(jnp.int32, (_B, _N), 1)
    M = jnp.max(sims, axis=1, keepdims=True)
    pos_sum_exp = jnp.sum(jnp.where(pos, jnp.exp(sims - M), 0.0),
                          axis=1, keepdims=True)

    def body(k, carry):
        cur, acc = carry
        m = jnp.max(cur, axis=1, keepdims=True)
        idx = jnp.min(jnp.where(cur == m, iota, _N), axis=1, keepdims=True)
        onehot = iota == idx
        val = jnp.sum(jnp.where(onehot, sims, 0.0), axis=1, keepdims=True)
        acc = acc + jnp.exp(val - M)
        cur = jnp.where(onehot, -1e30, cur)
        return cur, acc

    _, neg_sum_exp = jax.lax.fori_loop(
        0, _NH, body, (masked, jnp.zeros((_B, 1), jnp.float32)))
    lse = jnp.log(pos_sum_exp + neg_sum_exp) + M
    mean_pos = (jnp.sum(jnp.where(pos, sims, 0.0), axis=1, keepdims=True)
                / jnp.maximum(npos, 1.0))
    per = lse - mean_pos
    loss_ref[...] = jnp.sum(jnp.where(npos > 0.0, per, 0.0),
                            axis=0, keepdims=True) / _B

    d2p = jnp.maximum(1.0 - ip, 1e-8)
    cols = []
    for cj in range(_MC):
        cm = q == cj
        pc = jnp.sum(cm.astype(jnp.float32), axis=1, keepdims=True)
        kc = jnp.minimum(float(_NK), pc)
        cur = jnp.where(cm, d2p, 1e9)
        acc = jnp.zeros((_B, 1), jnp.float32)
        for k in range(_NK):
            m = jnp.min(cur, axis=1, keepdims=True)
            idx = jnp.min(jnp.where(cur == m, iota, _N), axis=1, keepdims=True)
            onehot = iota == idx
            acc = acc + jnp.where(kc > float(k), m, 0.0)
            cur = jnp.where(onehot, 2e9, cur)
        cols.append(acc / jnp.maximum(kc, 1.0))
    rowmeans_ref[...] = jnp.concatenate(cols, axis=1)


def _mmd_rbf(x, mask_x, y, mask_y):
    n = mask_x.sum().astype(jnp.int32)
    m = mask_y.sum().astype(jnp.int32)
    fx = mask_x.astype(jnp.float32)
    fy = mask_y.astype(jnp.float32)
    xx = (x[:, None] - x[None, :]) ** 2
    yy = (y[:, None] - y[None, :]) ** 2
    xy = (x[:, None] - y[None, :]) ** 2
    mxx = fx[:, None] * fx[None, :]
    myy = fy[:, None] * fy[None, :]
    mxy = fx[:, None] * fy[None, :]
    denom = jnp.maximum(1, n * n - n + m * m - m).astype(jnp.float32)
    sigma = jnp.maximum((xx * mxx).sum() + (yy * myy).sum(), 1e-06) / denom
    sigma = jnp.maximum(sigma, 1e-06)
    nn = jnp.maximum(n * n, 1).astype(jnp.float32)
    mm = jnp.maximum(m * m, 1).astype(jnp.float32)
    nm = jnp.maximum(n * m, 1).astype(jnp.float32)
    res = ((mxx * jnp.exp(-xx / sigma)).sum() / nn
           + (myy * jnp.exp(-yy / sigma)).sum() / mm
           - 2.0 * (mxy * jnp.exp(-xy / sigma)).sum() / nm)
    return jnp.where((n >= 2) & (m >= 2), res, 0.0).astype(jnp.float32)


@jax.jit
def kernel(inputs, targets, cams, proxy, pids, cids, D_cam):
    targets = targets.astype(jnp.int32)
    cams = cams.astype(jnp.int32)
    pids = pids.astype(jnp.int32)
    cids = cids.astype(jnp.int32)

    loss_main, gram, rowmeans = pl.pallas_call(
        _acm_main_kernel,
        out_shape=[
            jax.ShapeDtypeStruct((1, 1), jnp.float32),
            jax.ShapeDtypeStruct((_B, _B), jnp.float32),
            jax.ShapeDtypeStruct((_B, _MC), jnp.float32),
        ],
    )(inputs, targets.reshape(_B, 1), cams.reshape(_B, 1), proxy,
      pids.reshape(1, _N), cids.reshape(1, _N))
    loss = loss_main[0, 0]

    cam_range = jnp.arange(_MC)
    counts = jnp.sum(cams[None, :] == cam_range[:, None], axis=1)
    present = counts > 0
    has2 = jnp.sum(present) >= 2
    rank = jnp.cumsum(present.astype(jnp.int32)) - present.astype(jnp.int32)
    rank_or_neg = jnp.where(present, rank, -1)
    pcount = jnp.sum(cids[None, :] == cam_range[:, None], axis=1)

    pw2 = jnp.maximum(2.0 - 2.0 * gram, 0.0)
    ii, jj = np.triu_indices(_B, 1)
    dpair = jnp.sqrt(pw2[ii, jj])
    same = cams[ii] == cams[jj]

    xd = jnp.where(same, dpair, 0.0)
    yd = jnp.where(same, 0.0, dpair)
    loss_dis = _mmd_rbf(xd, same, yd, jnp.logical_not(same))

    D_work = D_cam
    for cc in range(_MC):
        mem = cams == cc
        pmask = (mem[ii] & mem[jj]).astype(jnp.float32)
        pc = pmask.sum()
        intra_mean = (pmask * dpair).sum() / jnp.maximum(pc, 1.0)
        r = jnp.maximum(rank_or_neg[cc], 0)
        old = D_work[r, r]
        newv = _GAMMA * old + (1.0 - _GAMMA) * jnp.maximum(intra_mean, 1e-06)
        D_work = D_work.at[r, r].set(jnp.where(counts[cc] >= 2, newv, old))
    for ci in range(_MC):
        memi = (cams == ci).astype(jnp.float32)
        for cj in range(_MC):
            if ci == cj:
                continue
            mean_d = ((rowmeans[:, cj] * memi).sum()
                      / jnp.maximum(counts[ci], 1).astype(jnp.float32))
            ri = jnp.maximum(rank_or_neg[ci], 0)
            rj = jnp.maximum(rank_or_neg[cj], 0)
            cond = present[ci] & present[cj] & (pcount[cj] > 0)
            old = D_work[ri, rj]
            newv = _GAMMA * old + (1.0 - _GAMMA) * jnp.maximum(mean_d, 1e-06)
            D_work = D_work.at[ri, rj].set(jnp.where(cond, newv, old))
    D = jnp.maximum(D_work, 1e-06)

    dist_raw = jnp.maximum(1.0 - gram, 1e-08)
    cam_idx_arr = rank_or_neg[cams]
    pos_mask_b = (targets[None, :] == targets[:, None]) & (
        cams[None, :] != cams[:, None])
    neg_mask_b = targets[None, :] != targets[:, None]
    hard_pos = jnp.argmin(jnp.where(pos_mask_b, dist_raw, jnp.inf), axis=1)
    hard_neg = jnp.argmax(jnp.where(neg_mask_b, dist_raw, -jnp.inf), axis=1)
    ic = cam_idx_arr
    ic_s = jnp.maximum(ic, 0)
    jc_pos = cam_idx_arr[hard_pos]
    jc_pos = jnp.where(jc_pos < 0, ic_s, jc_pos)
    jc_neg = cam_idx_arr[hard_neg]
    jc_neg = jnp.where(jc_neg < 0, ic_s, jc_neg)
    scale_pos = jnp.clip(D[ic_s, ic_s] / D[ic_s, jc_pos], 0.1, 10.0)
    scale_neg = jnp.clip(D[ic_s, ic_s] / D[ic_s, jc_neg], 0.1, 10.0)
    ar = jnp.arange(_B)
    d_pos = dist_raw[ar, hard_pos] * scale_pos
    d_neg = dist_raw[ar, hard_neg] * scale_neg
    valid = (ic >= 0) & (pos_mask_b.sum(axis=1) > 0) & (
        neg_mask_b.sum(axis=1) > 0)
    loss_ins = jnp.where(valid, jax.nn.relu(d_pos - d_neg + 0.2), 0.0).sum()
    loss_ins = loss_ins / jnp.maximum(valid.sum(), 1).astype(jnp.float32)

    loss = loss + jnp.where(has2, _LAM_DIS * loss_dis, 0.0)
    loss = loss + jnp.where(has2, _LAM_INS * loss_ins, 0.0)
    return loss


# speedup vs baseline: 2.5963x; 1.0097x over previous
import functools

import jax
import jax.numpy as jnp
import numpy as np
from jax.experimental import pallas as pl

_F = 256
_N = 8192
_NH = 50
_TEMP = 0.07
_LAM_DIS = 0.05
_LAM_INS = 0.05
_GAMMA = 0.9
_NK = 5
_MC = 8
_B = 64


def _acm_main_kernel(x_ref, t_ref, c_ref, proxy_ref, p_ref, q_ref,
                     loss_ref, gram_ref, rowmeans_ref):
    x = x_ref[...]
    nrm = jnp.sqrt(jnp.sum(x * x, axis=1, keepdims=True))
    xn = x / jnp.maximum(nrm, 1e-12)
    proxy = proxy_ref[...]
    ip = jax.lax.dot_general(xn, proxy, (((1,), (1,)), ((), ())),
                             preferred_element_type=jnp.float32)
    gram = jax.lax.dot_general(xn, xn, (((1,), (1,)), ((), ())),
                               preferred_element_type=jnp.float32)
    gram_ref[...] = gram

    t = t_ref[...]
    c = c_ref[...]
    p = p_ref[...]
    q = q_ref[...]

    sims = ip * (1.0 / _TEMP)
    pos = (t == p) & (c != q)
    neg = t != p
    npos = jnp.sum(pos.astype(jnp.float32), axis=1, keepdims=True)
    masked = sims + jnp.where(neg, 0.0, -9999999.0)

    iota = jax.lax.broadcasted_iota(jnp.int32, (_B, _N), 1)
    M = jnp.max(sims, axis=1, keepdims=True)
    pos_sum_exp = jnp.sum(jnp.where(pos, jnp.exp(sims - M), 0.0),
                          axis=1, keepdims=True)

    def body(k, carry):
        cur, acc, rem = carry
        m = jnp.max(cur, axis=1, keepdims=True)
        eq = cur == m
        cnt = jnp.sum(eq.astype(jnp.float32), axis=1, keepdims=True)
        sval = jnp.max(jnp.where(eq, sims, -1e30), axis=1, keepdims=True)
        take = jnp.minimum(cnt, rem)
        acc = acc + take * jnp.exp(sval - M)
        rem = rem - take
        cur = jnp.where(eq, -1e30, cur)
        return cur, acc, rem

    _, neg_sum_exp, _ = jax.lax.fori_loop(
        0, _NH, body, (masked, jnp.zeros((_B, 1), jnp.float32),
                       jnp.full((_B, 1), float(_NH), jnp.float32)))
    lse = jnp.log(pos_sum_exp + neg_sum_exp) + M
    mean_pos = (jnp.sum(jnp.where(pos, sims, 0.0), axis=1, keepdims=True)
                / jnp.maximum(npos, 1.0))
    per = lse - mean_pos
    loss_ref[...] = jnp.sum(jnp.where(npos > 0.0, per, 0.0),
                            axis=0, keepdims=True) / _B

    d2p = jnp.maximum(1.0 - ip, 1e-8)
    cols = []
    for cj in range(_MC):
        cm = q == cj
        pc = jnp.sum(cm.astype(jnp.float32), axis=1, keepdims=True)
        kc = jnp.minimum(float(_NK), pc)
        cur = jnp.where(cm, d2p, 1e9)
        acc = jnp.zeros((_B, 1), jnp.float32)
        rem = kc + jnp.zeros((_B, 1), jnp.float32)
        for k in range(_NK):
            m = jnp.min(cur, axis=1, keepdims=True)
            eq = cur == m
            cnt = jnp.sum(eq.astype(jnp.float32), axis=1, keepdims=True)
            take = jnp.minimum(cnt, rem)
            acc = acc + take * m
            rem = rem - take
            cur = jnp.where(eq, 2e9, cur)
        cols.append(acc / jnp.maximum(kc, 1.0))
    rowmeans_ref[...] = jnp.concatenate(cols, axis=1)


def _mmd_rbf(x, mask_x, y, mask_y):
    n = mask_x.sum().astype(jnp.int32)
    m = mask_y.sum().astype(jnp.int32)
    fx = mask_x.astype(jnp.float32)
    fy = mask_y.astype(jnp.float32)
    xx = (x[:, None] - x[None, :]) ** 2
    yy = (y[:, None] - y[None, :]) ** 2
    xy = (x[:, None] - y[None, :]) ** 2
    mxx = fx[:, None] * fx[None, :]
    myy = fy[:, None] * fy[None, :]
    mxy = fx[:, None] * fy[None, :]
    denom = jnp.maximum(1, n * n - n + m * m - m).astype(jnp.float32)
    sigma = jnp.maximum((xx * mxx).sum() + (yy * myy).sum(), 1e-06) / denom
    sigma = jnp.maximum(sigma, 1e-06)
    nn = jnp.maximum(n * n, 1).astype(jnp.float32)
    mm = jnp.maximum(m * m, 1).astype(jnp.float32)
    nm = jnp.maximum(n * m, 1).astype(jnp.float32)
    res = ((mxx * jnp.exp(-xx / sigma)).sum() / nn
           + (myy * jnp.exp(-yy / sigma)).sum() / mm
           - 2.0 * (mxy * jnp.exp(-xy / sigma)).sum() / nm)
    return jnp.where((n >= 2) & (m >= 2), res, 0.0).astype(jnp.float32)


@jax.jit
def kernel(inputs, targets, cams, proxy, pids, cids, D_cam):
    targets = targets.astype(jnp.int32)
    cams = cams.astype(jnp.int32)
    pids = pids.astype(jnp.int32)
    cids = cids.astype(jnp.int32)

    loss_main, gram, rowmeans = pl.pallas_call(
        _acm_main_kernel,
        out_shape=[
            jax.ShapeDtypeStruct((1, 1), jnp.float32),
            jax.ShapeDtypeStruct((_B, _B), jnp.float32),
            jax.ShapeDtypeStruct((_B, _MC), jnp.float32),
        ],
    )(inputs, targets.reshape(_B, 1), cams.reshape(_B, 1), proxy,
      pids.reshape(1, _N), cids.reshape(1, _N))
    loss = loss_main[0, 0]

    cam_range = jnp.arange(_MC)
    counts = jnp.sum(cams[None, :] == cam_range[:, None], axis=1)
    present = counts > 0
    has2 = jnp.sum(present) >= 2
    rank = jnp.cumsum(present.astype(jnp.int32)) - present.astype(jnp.int32)
    rank_or_neg = jnp.where(present, rank, -1)
    pcount = jnp.sum(cids[None, :] == cam_range[:, None], axis=1)

    pw2 = jnp.maximum(2.0 - 2.0 * gram, 0.0)
    ii, jj = np.triu_indices(_B, 1)
    dpair = jnp.sqrt(pw2[ii, jj])
    same = cams[ii] == cams[jj]

    xd = jnp.where(same, dpair, 0.0)
    yd = jnp.where(same, 0.0, dpair)
    loss_dis = _mmd_rbf(xd, same, yd, jnp.logical_not(same))

    D_work = D_cam
    for cc in range(_MC):
        mem = cams == cc
        pmask = (mem[ii] & mem[jj]).astype(jnp.float32)
        pc = pmask.sum()
        intra_mean = (pmask * dpair).sum() / jnp.maximum(pc, 1.0)
        r = jnp.maximum(rank_or_neg[cc], 0)
        old = D_work[r, r]
        newv = _GAMMA * old + (1.0 - _GAMMA) * jnp.maximum(intra_mean, 1e-06)
        D_work = D_work.at[r, r].set(jnp.where(counts[cc] >= 2, newv, old))
    for ci in range(_MC):
        memi = (cams == ci).astype(jnp.float32)
        for cj in range(_MC):
            if ci == cj:
                continue
            mean_d = ((rowmeans[:, cj] * memi).sum()
                      / jnp.maximum(counts[ci], 1).astype(jnp.float32))
            ri = jnp.maximum(rank_or_neg[ci], 0)
            rj = jnp.maximum(rank_or_neg[cj], 0)
            cond = present[ci] & present[cj] & (pcount[cj] > 0)
            old = D_work[ri, rj]
            newv = _GAMMA * old + (1.0 - _GAMMA) * jnp.maximum(mean_d, 1e-06)
            D_work = D_work.at[ri, rj].set(jnp.where(cond, newv, old))
    D = jnp.maximum(D_work, 1e-06)

    dist_raw = jnp.maximum(1.0 - gram, 1e-08)
    cam_idx_arr = rank_or_neg[cams]
    pos_mask_b = (targets[None, :] == targets[:, None]) & (
        cams[None, :] != cams[:, None])
    neg_mask_b = targets[None, :] != targets[:, None]
    hard_pos = jnp.argmin(jnp.where(pos_mask_b, dist_raw, jnp.inf), axis=1)
    hard_neg = jnp.argmax(jnp.where(neg_mask_b, dist_raw, -jnp.inf), axis=1)
    ic = cam_idx_arr
    ic_s = jnp.maximum(ic, 0)
    jc_pos = cam_idx_arr[hard_pos]
    jc_pos = jnp.where(jc_pos < 0, ic_s, jc_pos)
    jc_neg = cam_idx_arr[hard_neg]
    jc_neg = jnp.where(jc_neg < 0, ic_s, jc_neg)
    scale_pos = jnp.clip(D[ic_s, ic_s] / D[ic_s, jc_pos], 0.1, 10.0)
    scale_neg = jnp.clip(D[ic_s, ic_s] / D[ic_s, jc_neg], 0.1, 10.0)
    ar = jnp.arange(_B)
    d_pos = dist_raw[ar, hard_pos] * scale_pos
    d_neg = dist_raw[ar, hard_neg] * scale_neg
    valid = (ic >= 0) & (pos_mask_b.sum(axis=1) > 0) & (
        neg_mask_b.sum(axis=1) > 0)
    loss_ins = jnp.where(valid, jax.nn.relu(d_pos - d_neg + 0.2), 0.0).sum()
    loss_ins = loss_ins / jnp.maximum(valid.sum(), 1).astype(jnp.float32)

    loss = loss + jnp.where(has2, _LAM_DIS * loss_dis, 0.0)
    loss = loss + jnp.where(has2, _LAM_INS * loss_ins, 0.0)
    return loss
